# baseline (speedup 1.0000x reference)
import functools

import jax
import jax.numpy as jnp
from jax import lax
from jax.experimental import pallas as pl
from jax.experimental.pallas import tpu as pltpu
from jax.experimental.pallas import tpu_sc as plsc

DIM = 64
CHUNK = 128


def kernel(xinput, weights):
    B, L = xinput.shape
    info = plsc.get_sparse_core_info()
    nw = info.num_cores * info.num_subcores
    nchunks = B // CHUNK
    cpw = nchunks // nw

    @functools.partial(
        pl.kernel,
        mesh=plsc.VectorSubcoreMesh(core_axis_name="c", subcore_axis_name="s"),
        out_type=jax.ShapeDtypeStruct((B, DIM), jnp.float32),
        scratch_types=[
            pltpu.VMEM((CHUNK, L), jnp.int32),
            pltpu.VMEM((cpw, L, CHUNK), jnp.int32),
            pltpu.VMEM((cpw, CHUNK, DIM), jnp.float32),
        ]
        + [pltpu.SemaphoreType.DMA] * cpw,
        compiler_params=pltpu.CompilerParams(
            use_tc_tiling_on_sc=False, needs_layout_passes=False
        ),
    )
    def sc_kernel(idx_hbm, table_hbm, out_hbm, nat_v, idx_v, acc, *sems):
        wid = lax.axis_index("s") * info.num_cores + lax.axis_index("c")

        zero = jnp.zeros((16,), jnp.float32)
        for c in range(cpw):
            pltpu.sync_copy(idx_hbm.at[pl.ds((wid * cpw + c) * CHUNK, CHUNK)], nat_v)

            row_ids = [
                jax.lax.iota(jnp.int32, 16) + (16 * g) for g in range(CHUNK // 16)
            ]

            def transpose_l(l, _, c=c):
                col = jnp.full((16,), l, jnp.int32)
                for g in range(CHUNK // 16):
                    v = plsc.load_gather(nat_v, [row_ids[g], col])
                    idx_v[c, l, pl.ds(16 * g, 16)] = v
                return 0

            lax.fori_loop(0, L, transpose_l, 0)

            def zero_row(j, _, c=c):
                for d in range(DIM // 16):
                    acc[c, j, pl.ds(16 * d, 16)] = zero
                return 0

            lax.fori_loop(0, CHUNK, zero_row, 0)

        for c in range(cpw):

            def fire(l, _, c=c):
                pltpu.async_copy(
                    table_hbm.at[idx_v.at[c, l]], acc.at[c], sems[c], add=True
                )
                return 0

            lax.fori_loop(0, L, fire, 0)

        for c in range(cpw):

            def drain(l, _, c=c):
                pltpu.make_async_copy(
                    table_hbm.at[idx_v.at[c, 0]], acc.at[c], sems[c]
                ).wait()
                return 0

            lax.fori_loop(0, L, drain, 0)
            pltpu.sync_copy(acc.at[c], out_hbm.at[pl.ds((wid * cpw + c) * CHUNK, CHUNK)])

    return sc_kernel(xinput, weights)
